# baseline (speedup 1.0000x reference)
import functools

import jax
import jax.numpy as jnp
from jax.experimental import pallas as pl
from jax.experimental.pallas import tpu as pltpu

_KINEMATIC_COEF = 0.1
_NEG_INF = float("-inf")
_POS_INF = float("inf")


def _retrieve_body(qn_ref, kdb_ref, ql_ref, mlc_ref, ml_ref,
                   vals_ref, idx_ref,
                   r1v_ref, r1i_ref, r2v_ref, r2i_ref,
                   *, block_k, num_keys, num_blocks):
    k = pl.program_id(0)

    tail = num_keys - (num_blocks - 1) * block_k

    @pl.when(k == num_blocks - 1)
    def _zero_pad():
        kdb_ref[pl.ds(tail, block_k - tail), :] = jnp.zeros(
            (block_k - tail, kdb_ref.shape[1]), jnp.float32)

    kb = kdb_ref[...]
    knorm = jnp.sqrt(jnp.sum(kb * kb, axis=1, keepdims=True))
    kn = kb / jnp.maximum(knorm, 1e-8)

    semantic = jax.lax.dot_general(
        qn_ref[...], kn,
        dimension_numbers=(((1,), (1,)), ((), ())),
        preferred_element_type=jnp.float32,
    )

    ml = ml_ref[...]
    mlc = mlc_ref[...]
    ql = ql_ref[...]
    denom = jnp.maximum(mlc, ql)
    rel = jnp.abs(ml - ql) / denom
    score = semantic * jnp.exp(rel * (-_KINEMATIC_COEF))

    colf = jax.lax.broadcasted_iota(jnp.int32, (1, block_k), 1
                                    ).astype(jnp.float32)
    q_dim = score.shape[0]
    m1v = jnp.max(score, axis=1, keepdims=True)
    eq1 = score == m1v
    m1l = jnp.min(jnp.where(eq1, colf, _POS_INF), axis=1, keepdims=True)
    masked = jnp.where(eq1, _NEG_INF, score)
    m2v = jnp.max(masked, axis=1, keepdims=True)
    m2l = jnp.min(jnp.where(masked == m2v, colf, _POS_INF), axis=1,
                  keepdims=True)
    base = k * block_k
    m1i = m1l.astype(jnp.int32) + base
    m2i = m2l.astype(jnp.int32) + base

    fresh = k == 0
    r1v = jnp.where(fresh, _NEG_INF, r1v_ref[...])
    r2v = jnp.where(fresh, _NEG_INF, r2v_ref[...])
    r1i = jnp.where(fresh, 0, r1i_ref[...])
    r2i = jnp.where(fresh, 0, r2i_ref[...])
    first_run = r1v >= m1v
    n1v = jnp.where(first_run, r1v, m1v)
    n1i = jnp.where(first_run, r1i, m1i)
    cr = r2v >= m1v
    cb = r1v >= m2v
    n2v = jnp.where(first_run, jnp.where(cr, r2v, m1v),
                    jnp.where(cb, r1v, m2v))
    n2i = jnp.where(first_run, jnp.where(cr, r2i, m1i),
                    jnp.where(cb, r1i, m2i))
    r1v_ref[...], r1i_ref[...] = n1v, n1i
    r2v_ref[...], r2i_ref[...] = n2v, n2i

    @pl.when(k == num_blocks - 1)
    def _finish():
        vals_ref[...] = jnp.concatenate([n1v, n2v], axis=1)
        idx_ref[...] = jnp.concatenate([n1i, n2i], axis=1)


def kernel(queries, keys_db, lengths, m_lengths):
    q_dim, d = queries.shape
    num_keys = keys_db.shape[0]
    block_k = 6144
    num_blocks = pl.cdiv(num_keys, block_k)

    qnorm = jnp.sqrt(jnp.sum(queries * queries, axis=1, keepdims=True))
    qn = queries / jnp.maximum(qnorm, 1e-8)
    ql = lengths.astype(jnp.float32).reshape(q_dim, 1)
    padded_keys = num_blocks * block_k
    ml = jnp.pad(m_lengths.astype(jnp.float32), (0, padded_keys - num_keys),
                 constant_values=_NEG_INF).reshape(1, padded_keys)
    mlc = jnp.maximum(ml, 1.0)

    body = functools.partial(_retrieve_body, block_k=block_k,
                             num_keys=num_keys, num_blocks=num_blocks)

    vals, idx = pl.pallas_call(
        body,
        grid=(num_blocks,),
        in_specs=[
            pl.BlockSpec((q_dim, d), lambda k: (0, 0)),
            pl.BlockSpec((block_k, d), lambda k: (k, 0)),
            pl.BlockSpec((q_dim, 1), lambda k: (0, 0)),
            pl.BlockSpec((1, block_k), lambda k: (0, k)),
            pl.BlockSpec((1, block_k), lambda k: (0, k)),
        ],
        out_specs=[
            pl.BlockSpec((q_dim, 2), lambda k: (0, 0)),
            pl.BlockSpec((q_dim, 2), lambda k: (0, 0)),
        ],
        out_shape=[
            jax.ShapeDtypeStruct((q_dim, 2), jnp.float32),
            jax.ShapeDtypeStruct((q_dim, 2), jnp.int32),
        ],
        scratch_shapes=[
            pltpu.VMEM((q_dim, 1), jnp.float32),
            pltpu.VMEM((q_dim, 1), jnp.int32),
            pltpu.VMEM((q_dim, 1), jnp.float32),
            pltpu.VMEM((q_dim, 1), jnp.int32),
        ],
        compiler_params=pltpu.CompilerParams(
            dimension_semantics=("arbitrary",),
        ),
    )(qn, keys_db, ql, mlc, ml)
    return vals, idx


# speedup vs baseline: 1.0171x; 1.0171x over previous
import functools

import jax
import jax.numpy as jnp
from jax.experimental import pallas as pl
from jax.experimental.pallas import tpu as pltpu

_KINEMATIC_COEF = 0.1
_NEG_INF = float("-inf")
_POS_INF = float("inf")


def _retrieve_body(qn_ref, kdb_ref, ql_ref, mlc_ref, ml_ref,
                   vals_ref, idx_ref,
                   r1v_ref, r1i_ref, r2v_ref, r2i_ref,
                   *, block_k, num_keys, num_blocks):
    k = pl.program_id(0)

    tail = num_keys - (num_blocks - 1) * block_k

    @pl.when(k == num_blocks - 1)
    def _zero_pad():
        kdb_ref[pl.ds(tail, block_k - tail), :] = jnp.zeros(
            (block_k - tail, kdb_ref.shape[1]), jnp.float32)

    kb = kdb_ref[...]
    knorm = jnp.sqrt(jnp.sum(kb * kb, axis=1, keepdims=True))
    kn = kb / jnp.maximum(knorm, 1e-8)

    semantic = jax.lax.dot_general(
        qn_ref[...], kn,
        dimension_numbers=(((1,), (1,)), ((), ())),
        preferred_element_type=jnp.float32,
    )

    ml = ml_ref[...]
    mlc = mlc_ref[...]
    ql = ql_ref[...]
    denom = jnp.maximum(mlc, ql)
    rel = jnp.abs(ml - ql) / denom
    score = semantic * jnp.exp(rel * (-_KINEMATIC_COEF))

    colf = jax.lax.broadcasted_iota(jnp.int32, (1, block_k), 1
                                    ).astype(jnp.float32)
    q_dim = score.shape[0]
    m1v = jnp.max(score, axis=1, keepdims=True)
    eq1 = score == m1v
    m1l = jnp.min(jnp.where(eq1, colf, _POS_INF), axis=1, keepdims=True)
    masked = jnp.where(eq1, _NEG_INF, score)
    m2v = jnp.max(masked, axis=1, keepdims=True)
    m2l = jnp.min(jnp.where(masked == m2v, colf, _POS_INF), axis=1,
                  keepdims=True)
    base = k * block_k
    m1i = m1l.astype(jnp.int32) + base
    m2i = m2l.astype(jnp.int32) + base

    fresh = k == 0
    r1v = jnp.where(fresh, _NEG_INF, r1v_ref[...])
    r2v = jnp.where(fresh, _NEG_INF, r2v_ref[...])
    r1i = jnp.where(fresh, 0, r1i_ref[...])
    r2i = jnp.where(fresh, 0, r2i_ref[...])
    first_run = r1v >= m1v
    n1v = jnp.where(first_run, r1v, m1v)
    n1i = jnp.where(first_run, r1i, m1i)
    cr = r2v >= m1v
    cb = r1v >= m2v
    n2v = jnp.where(first_run, jnp.where(cr, r2v, m1v),
                    jnp.where(cb, r1v, m2v))
    n2i = jnp.where(first_run, jnp.where(cr, r2i, m1i),
                    jnp.where(cb, r1i, m2i))
    r1v_ref[...], r1i_ref[...] = n1v, n1i
    r2v_ref[...], r2i_ref[...] = n2v, n2i

    @pl.when(k == num_blocks - 1)
    def _finish():
        vals_ref[...] = jnp.concatenate([n1v, n2v], axis=1)
        idx_ref[...] = jnp.concatenate([n1i, n2i], axis=1)


def kernel(queries, keys_db, lengths, m_lengths):
    q_dim, d = queries.shape
    num_keys = keys_db.shape[0]
    block_k = 4096
    num_blocks = pl.cdiv(num_keys, block_k)

    qnorm = jnp.sqrt(jnp.sum(queries * queries, axis=1, keepdims=True))
    qn = queries / jnp.maximum(qnorm, 1e-8)
    ql = lengths.astype(jnp.float32).reshape(q_dim, 1)
    padded_keys = num_blocks * block_k
    ml = jnp.pad(m_lengths.astype(jnp.float32), (0, padded_keys - num_keys),
                 constant_values=_NEG_INF).reshape(1, padded_keys)
    mlc = jnp.maximum(ml, 1.0)

    body = functools.partial(_retrieve_body, block_k=block_k,
                             num_keys=num_keys, num_blocks=num_blocks)

    vals, idx = pl.pallas_call(
        body,
        grid=(num_blocks,),
        in_specs=[
            pl.BlockSpec((q_dim, d), lambda k: (0, 0)),
            pl.BlockSpec((block_k, d), lambda k: (k, 0)),
            pl.BlockSpec((q_dim, 1), lambda k: (0, 0)),
            pl.BlockSpec((1, block_k), lambda k: (0, k)),
            pl.BlockSpec((1, block_k), lambda k: (0, k)),
        ],
        out_specs=[
            pl.BlockSpec((q_dim, 2), lambda k: (0, 0)),
            pl.BlockSpec((q_dim, 2), lambda k: (0, 0)),
        ],
        out_shape=[
            jax.ShapeDtypeStruct((q_dim, 2), jnp.float32),
            jax.ShapeDtypeStruct((q_dim, 2), jnp.int32),
        ],
        scratch_shapes=[
            pltpu.VMEM((q_dim, 1), jnp.float32),
            pltpu.VMEM((q_dim, 1), jnp.int32),
            pltpu.VMEM((q_dim, 1), jnp.float32),
            pltpu.VMEM((q_dim, 1), jnp.int32),
        ],
        compiler_params=pltpu.CompilerParams(
            dimension_semantics=("arbitrary",),
        ),
    )(qn, keys_db, ql, mlc, ml)
    return vals, idx
